# Initial kernel scaffold; baseline (speedup 1.0000x reference)
import jax
import jax.numpy as jnp
from jax.experimental import pallas as pl


def kernel(flow, clusters):
    raise NotImplementedError("write your pallas kernel here")


# speedup vs baseline: 5.1561x; 5.1561x over previous
import functools

import jax
import jax.numpy as jnp
from jax import lax
from jax.experimental import pallas as pl
from jax.experimental.pallas import tpu as pltpu
from jax.experimental.pallas import tpu_sc as plsc

N_POINTS = 100000
N_BINS = 512
NC = 2
NS = 16
L = 16

NW = NC * NS
CHUNK4 = 3136
N_PAD = NW * CHUNK4
CHUNK1 = N_PAD // NS
NBP = 576
ACC = 4 * NBP
COLCH = ACC // NS


def _rsqrt(x):
    xb = plsc.bitcast(x, jnp.int32)
    y = plsc.bitcast(jnp.int32(0x5F3759DF) - lax.shift_right_logical(xb, 1),
                     jnp.float32)
    hx = x * 0.5
    for _ in range(3):
        y = y * (1.5 - hx * y * y)
    return y


def _floxels_kernel(flow_hbm, clus_hbm, out_hbm,
                    flow_v, clus_v, hist_v, colchunk_v, combchunk_v, out_v,
                    part_sh, comb_sh):
    c = lax.axis_index("c")
    s = lax.axis_index("s")
    iota = lax.iota(jnp.int32, L)
    zeros16 = jnp.zeros((L,), jnp.float32)
    ones16 = jnp.ones((L,), jnp.float32)

    def zero_body(i, _):
        hist_v[pl.ds(i * L, L)] = zeros16
        return 0
    lax.fori_loop(0, ACC // L, zero_body, 0)

    base1 = s * CHUNK1
    pltpu.sync_copy(flow_hbm.at[pl.ds(base1 * 3, CHUNK1 * 3)], flow_v)
    pltpu.sync_copy(clus_hbm.at[pl.ds(base1, CHUNK1)], clus_v)

    def p1_body(i, _):
        cl = clus_v[pl.ds(i * L, L)]
        i4 = cl * 4
        pid3 = i * (3 * L) + iota * 3
        x = plsc.load_gather(flow_v, [pid3])
        y = plsc.load_gather(flow_v, [pid3 + 1])
        z = plsc.load_gather(flow_v, [pid3 + 2])
        plsc.addupdate_scatter(hist_v, [i4], x)
        plsc.addupdate_scatter(hist_v, [i4 + 1], y)
        plsc.addupdate_scatter(hist_v, [i4 + 2], z)
        plsc.addupdate_scatter(hist_v, [i4 + 3], ones16)
        return 0
    lax.fori_loop(0, CHUNK1 // L, p1_body, 0)

    pltpu.sync_copy(hist_v, part_sh.at[pl.ds(s * ACC, ACC)])
    plsc.subcore_barrier()
    for l in range(NS):
        pltpu.sync_copy(part_sh.at[pl.ds(l * ACC + s * COLCH, COLCH)],
                        colchunk_v.at[pl.ds(l * COLCH, COLCH)])

    def p2_body(j, _):
        acc = colchunk_v[pl.ds(j * L, L)]
        for l in range(1, NS):
            acc = acc + colchunk_v[pl.ds(l * COLCH + j * L, L)]
        combchunk_v[pl.ds(j * L, L)] = acc
        return 0
    lax.fori_loop(0, COLCH // L, p2_body, 0)

    pltpu.sync_copy(combchunk_v, comb_sh.at[pl.ds(s * COLCH, COLCH)])
    plsc.subcore_barrier()
    pltpu.sync_copy(comb_sh, hist_v)

    def p3_body(j, _):
        base = j * L
        num = hist_v[pl.ds(base, L)]
        den = plsc.load_gather(hist_v, [base + (iota | 3)])
        den = jnp.maximum(den, 1.0)
        hist_v[pl.ds(base, L)] = num / den
        return 0
    lax.fori_loop(0, ACC // L, p3_body, 0)

    w = c * NS + s
    base4 = w * CHUNK4
    pltpu.sync_copy(flow_hbm.at[pl.ds(base4 * 3, CHUNK4 * 3)],
                    flow_v.at[pl.ds(0, CHUNK4 * 3)])
    pltpu.sync_copy(clus_hbm.at[pl.ds(base4, CHUNK4)],
                    clus_v.at[pl.ds(0, CHUNK4)])

    def p4_body(i, _):
        cl = clus_v[pl.ds(i * L, L)]
        i4 = cl * 4
        pid3 = i * (3 * L) + iota * 3
        x = plsc.load_gather(flow_v, [pid3])
        y = plsc.load_gather(flow_v, [pid3 + 1])
        z = plsc.load_gather(flow_v, [pid3 + 2])
        dx = x - plsc.load_gather(hist_v, [i4])
        dy = y - plsc.load_gather(hist_v, [i4 + 1])
        dz = z - plsc.load_gather(hist_v, [i4 + 2])
        ss = dx * dx + dy * dy + dz * dz
        out_v[pl.ds(i * L, L)] = ss * _rsqrt(jnp.maximum(ss, 1e-30))
        return 0
    lax.fori_loop(0, CHUNK4 // L, p4_body, 0)

    pltpu.sync_copy(out_v, out_hbm.at[pl.ds(base4, CHUNK4)])


@jax.jit
def kernel(flow, clusters):
    pad = N_PAD - N_POINTS
    flow_p = jnp.pad(flow, ((0, pad), (0, 0))).reshape(-1)
    clus_p = jnp.pad(clusters, (0, pad), constant_values=N_BINS)

    mesh = plsc.VectorSubcoreMesh(core_axis_name="c", subcore_axis_name="s")
    run = pl.kernel(
        _floxels_kernel,
        mesh=mesh,
        compiler_params=pltpu.CompilerParams(needs_layout_passes=False),
        out_type=jax.ShapeDtypeStruct((N_PAD,), jnp.float32),
        scratch_types=[
            pltpu.VMEM((CHUNK1 * 3,), jnp.float32),
            pltpu.VMEM((CHUNK1,), jnp.int32),
            pltpu.VMEM((ACC,), jnp.float32),
            pltpu.VMEM((NS * COLCH,), jnp.float32),
            pltpu.VMEM((COLCH,), jnp.float32),
            pltpu.VMEM((CHUNK4,), jnp.float32),
            pltpu.VMEM_SHARED((NS * ACC,), jnp.float32),
            pltpu.VMEM_SHARED((ACC,), jnp.float32),
        ],
    )
    out_p = run(flow_p, clus_p)
    return out_p[:N_POINTS]


# speedup vs baseline: 6.8821x; 1.3347x over previous
import jax
import jax.numpy as jnp
from jax import lax
from jax.experimental import pallas as pl
from jax.experimental.pallas import tpu as pltpu
from jax.experimental.pallas import tpu_sc as plsc

N_POINTS = 100000
N_BINS = 512
NC = 2
NS = 16
L = 16

NW = NC * NS
CHUNK4 = 3136
CHUNK4_LAST = N_POINTS - (NW - 1) * CHUNK4
CHUNK1 = 6272
CHUNK1_LAST = N_POINTS - (NS - 1) * CHUNK1
NBP = 576
ACC = 4 * NBP
COLCH = ACC // NS


def _rsqrt(x):
    xb = plsc.bitcast(x, jnp.int32)
    y = plsc.bitcast(jnp.int32(0x5F3759DF) - lax.shift_right_logical(xb, 1),
                     jnp.float32)
    hx = x * 0.5
    for _ in range(3):
        y = y * (1.5 - hx * y * y)
    return y


def _floxels_kernel(flow_hbm, clus_hbm, out_hbm,
                    flow_v, clus_v, hist_v, colchunk_v, combchunk_v, out_v,
                    part_sh, comb_sh):
    c = lax.axis_index("c")
    s = lax.axis_index("s")
    iota = lax.iota(jnp.int32, L)
    zeros16 = jnp.zeros((L,), jnp.float32)
    ones16 = jnp.ones((L,), jnp.float32)

    def zero_body(i, _):
        hist_v[pl.ds(i * L, L)] = zeros16
        return 0
    lax.fori_loop(0, ACC // L, zero_body, 0)

    def phase1(base, npts):
        pltpu.sync_copy(flow_hbm.at[pl.ds(base * 3, npts * 3)],
                        flow_v.at[pl.ds(0, npts * 3)])
        pltpu.sync_copy(clus_hbm.at[pl.ds(base, npts)],
                        clus_v.at[pl.ds(0, npts)])

        def p1_body(i, _):
            cl = clus_v[pl.ds(i * L, L)]
            i4 = cl * 4
            pid3 = i * (3 * L) + iota * 3
            x = plsc.load_gather(flow_v, [pid3])
            y = plsc.load_gather(flow_v, [pid3 + 1])
            z = plsc.load_gather(flow_v, [pid3 + 2])
            plsc.addupdate_scatter(hist_v, [i4], x)
            plsc.addupdate_scatter(hist_v, [i4 + 1], y)
            plsc.addupdate_scatter(hist_v, [i4 + 2], z)
            plsc.addupdate_scatter(hist_v, [i4 + 3], ones16)
            return 0
        lax.fori_loop(0, npts // L, p1_body, 0)

    pl.when(s < NS - 1)(lambda: phase1(s * CHUNK1, CHUNK1))
    pl.when(s == NS - 1)(lambda: phase1((NS - 1) * CHUNK1, CHUNK1_LAST))

    pltpu.sync_copy(hist_v, part_sh.at[pl.ds(s * ACC, ACC)])
    plsc.subcore_barrier()
    for l in range(NS):
        pltpu.sync_copy(part_sh.at[pl.ds(l * ACC + s * COLCH, COLCH)],
                        colchunk_v.at[pl.ds(l * COLCH, COLCH)])

    def p2_body(j, _):
        acc = colchunk_v[pl.ds(j * L, L)]
        for l in range(1, NS):
            acc = acc + colchunk_v[pl.ds(l * COLCH + j * L, L)]
        combchunk_v[pl.ds(j * L, L)] = acc
        return 0
    lax.fori_loop(0, COLCH // L, p2_body, 0)

    pltpu.sync_copy(combchunk_v, comb_sh.at[pl.ds(s * COLCH, COLCH)])
    plsc.subcore_barrier()
    pltpu.sync_copy(comb_sh, hist_v)

    def p3_body(j, _):
        base = j * L
        num = hist_v[pl.ds(base, L)]
        den = plsc.load_gather(hist_v, [base + (iota | 3)])
        den = jnp.maximum(den, 1.0)
        hist_v[pl.ds(base, L)] = num / den
        return 0
    lax.fori_loop(0, ACC // L, p3_body, 0)

    def phase4(base, npts):
        pltpu.sync_copy(flow_hbm.at[pl.ds(base * 3, npts * 3)],
                        flow_v.at[pl.ds(0, npts * 3)])
        pltpu.sync_copy(clus_hbm.at[pl.ds(base, npts)],
                        clus_v.at[pl.ds(0, npts)])

        def p4_body(i, _):
            cl = clus_v[pl.ds(i * L, L)]
            i4 = cl * 4
            pid3 = i * (3 * L) + iota * 3
            x = plsc.load_gather(flow_v, [pid3])
            y = plsc.load_gather(flow_v, [pid3 + 1])
            z = plsc.load_gather(flow_v, [pid3 + 2])
            dx = x - plsc.load_gather(hist_v, [i4])
            dy = y - plsc.load_gather(hist_v, [i4 + 1])
            dz = z - plsc.load_gather(hist_v, [i4 + 2])
            ss = dx * dx + dy * dy + dz * dz
            out_v[pl.ds(i * L, L)] = ss * _rsqrt(jnp.maximum(ss, 1e-30))
            return 0
        lax.fori_loop(0, npts // L, p4_body, 0)

        pltpu.sync_copy(out_v.at[pl.ds(0, npts)], out_hbm.at[pl.ds(base, npts)])

    w = c * NS + s
    pl.when(w < NW - 1)(lambda: phase4(w * CHUNK4, CHUNK4))
    pl.when(w == NW - 1)(lambda: phase4((NW - 1) * CHUNK4, CHUNK4_LAST))


@jax.jit
def kernel(flow, clusters):
    flow_flat = flow.reshape(-1)

    mesh = plsc.VectorSubcoreMesh(core_axis_name="c", subcore_axis_name="s")
    run = pl.kernel(
        _floxels_kernel,
        mesh=mesh,
        compiler_params=pltpu.CompilerParams(needs_layout_passes=False),
        out_type=jax.ShapeDtypeStruct((N_POINTS,), jnp.float32),
        scratch_types=[
            pltpu.VMEM((CHUNK1 * 3,), jnp.float32),
            pltpu.VMEM((CHUNK1,), jnp.int32),
            pltpu.VMEM((ACC,), jnp.float32),
            pltpu.VMEM((NS * COLCH,), jnp.float32),
            pltpu.VMEM((COLCH,), jnp.float32),
            pltpu.VMEM((CHUNK4,), jnp.float32),
            pltpu.VMEM_SHARED((NS * ACC,), jnp.float32),
            pltpu.VMEM_SHARED((ACC,), jnp.float32),
        ],
    )
    return run(flow_flat, clusters)


# speedup vs baseline: 14.1039x; 2.0494x over previous
import jax
import jax.numpy as jnp
from jax import lax
from jax.experimental import pallas as pl
from jax.experimental.pallas import tpu as pltpu
from jax.experimental.pallas import tpu_sc as plsc

N_POINTS = 100000
N_BINS = 512
NC = 2
NS = 16
L = 16

NW = NC * NS
CHUNK4 = 3136
CHUNK4_LAST = N_POINTS - (NW - 1) * CHUNK4
CHUNK1 = 6272
CHUNK1_LAST = N_POINTS - (NS - 1) * CHUNK1
NBP = 576
ACC = 4 * NBP
COLCH = ACC // NS


def _rsqrt(x):
    xb = plsc.bitcast(x, jnp.int32)
    y = plsc.bitcast(jnp.int32(0x5F3759DF) - lax.shift_right_logical(xb, 1),
                     jnp.float32)
    hx = x * 0.5
    for _ in range(3):
        y = y * (1.5 - hx * y * y)
    return y


def _floxels_kernel(fx_hbm, fy_hbm, fz_hbm, clus_hbm, out_hbm,
                    fx_v, fy_v, fz_v, clus_v, hist_v, colchunk_v,
                    combchunk_v, out_v, part_sh, comb_sh):
    c = lax.axis_index("c")
    s = lax.axis_index("s")
    iota = lax.iota(jnp.int32, L)
    zeros16 = jnp.zeros((L,), jnp.float32)
    ones16 = jnp.ones((L,), jnp.float32)

    def zero_body(i, _):
        hist_v[pl.ds(i * L, L)] = zeros16
        return 0
    lax.fori_loop(0, ACC // L, zero_body, 0)

    def load_slices(base, npts):
        pltpu.sync_copy(fx_hbm.at[pl.ds(base, npts)], fx_v.at[pl.ds(0, npts)])
        pltpu.sync_copy(fy_hbm.at[pl.ds(base, npts)], fy_v.at[pl.ds(0, npts)])
        pltpu.sync_copy(fz_hbm.at[pl.ds(base, npts)], fz_v.at[pl.ds(0, npts)])
        pltpu.sync_copy(clus_hbm.at[pl.ds(base, npts)],
                        clus_v.at[pl.ds(0, npts)])

    def phase1(base, npts):
        load_slices(base, npts)

        def p1_body(i, _):
            b = i * L
            cl = clus_v[pl.ds(b, L)]
            i4 = cl * 4
            plsc.addupdate_scatter(hist_v, [i4], fx_v[pl.ds(b, L)])
            plsc.addupdate_scatter(hist_v, [i4 + 1], fy_v[pl.ds(b, L)])
            plsc.addupdate_scatter(hist_v, [i4 + 2], fz_v[pl.ds(b, L)])
            plsc.addupdate_scatter(hist_v, [i4 + 3], ones16)
            return 0
        lax.fori_loop(0, npts // L, p1_body, 0)

    pl.when(s < NS - 1)(lambda: phase1(s * CHUNK1, CHUNK1))
    pl.when(s == NS - 1)(lambda: phase1((NS - 1) * CHUNK1, CHUNK1_LAST))

    pltpu.sync_copy(hist_v, part_sh.at[pl.ds(s * ACC, ACC)])
    plsc.subcore_barrier()
    for l in range(NS):
        pltpu.sync_copy(part_sh.at[pl.ds(l * ACC + s * COLCH, COLCH)],
                        colchunk_v.at[pl.ds(l * COLCH, COLCH)])

    def p2_body(j, _):
        acc = colchunk_v[pl.ds(j * L, L)]
        for l in range(1, NS):
            acc = acc + colchunk_v[pl.ds(l * COLCH + j * L, L)]
        combchunk_v[pl.ds(j * L, L)] = acc
        return 0
    lax.fori_loop(0, COLCH // L, p2_body, 0)

    pltpu.sync_copy(combchunk_v, comb_sh.at[pl.ds(s * COLCH, COLCH)])
    plsc.subcore_barrier()
    pltpu.sync_copy(comb_sh, hist_v)

    def p3_body(j, _):
        base = j * L
        num = hist_v[pl.ds(base, L)]
        den = plsc.load_gather(hist_v, [base + (iota | 3)])
        den = jnp.maximum(den, 1.0)
        hist_v[pl.ds(base, L)] = num / den
        return 0
    lax.fori_loop(0, ACC // L, p3_body, 0)

    def phase4(base, npts):
        load_slices(base, npts)

        def p4_body(i, _):
            b = i * L
            cl = clus_v[pl.ds(b, L)]
            i4 = cl * 4
            dx = fx_v[pl.ds(b, L)] - plsc.load_gather(hist_v, [i4])
            dy = fy_v[pl.ds(b, L)] - plsc.load_gather(hist_v, [i4 + 1])
            dz = fz_v[pl.ds(b, L)] - plsc.load_gather(hist_v, [i4 + 2])
            ss = dx * dx + dy * dy + dz * dz
            out_v[pl.ds(b, L)] = ss * _rsqrt(jnp.maximum(ss, 1e-30))
            return 0
        lax.fori_loop(0, npts // L, p4_body, 0)

        pltpu.sync_copy(out_v.at[pl.ds(0, npts)], out_hbm.at[pl.ds(base, npts)])

    w = c * NS + s
    pl.when(w < NW - 1)(lambda: phase4(w * CHUNK4, CHUNK4))
    pl.when(w == NW - 1)(lambda: phase4((NW - 1) * CHUNK4, CHUNK4_LAST))


@jax.jit
def kernel(flow, clusters):
    fx = flow[:, 0]
    fy = flow[:, 1]
    fz = flow[:, 2]

    mesh = plsc.VectorSubcoreMesh(core_axis_name="c", subcore_axis_name="s")
    run = pl.kernel(
        _floxels_kernel,
        mesh=mesh,
        compiler_params=pltpu.CompilerParams(needs_layout_passes=False),
        out_type=jax.ShapeDtypeStruct((N_POINTS,), jnp.float32),
        scratch_types=[
            pltpu.VMEM((CHUNK1,), jnp.float32),
            pltpu.VMEM((CHUNK1,), jnp.float32),
            pltpu.VMEM((CHUNK1,), jnp.float32),
            pltpu.VMEM((CHUNK1,), jnp.int32),
            pltpu.VMEM((ACC,), jnp.float32),
            pltpu.VMEM((NS * COLCH,), jnp.float32),
            pltpu.VMEM((COLCH,), jnp.float32),
            pltpu.VMEM((CHUNK4,), jnp.float32),
            pltpu.VMEM_SHARED((NS * ACC,), jnp.float32),
            pltpu.VMEM_SHARED((ACC,), jnp.float32),
        ],
    )
    return run(fx, fy, fz, clusters)


# speedup vs baseline: 20.8546x; 1.4786x over previous
import jax
import jax.numpy as jnp
from jax import lax
from jax.experimental import pallas as pl
from jax.experimental.pallas import tpu as pltpu
from jax.experimental.pallas import tpu_sc as plsc

N_POINTS = 100000
N_BINS = 512
NC = 2
NS = 16
L = 16

CHUNK1 = 6272
CHUNK1_LAST = N_POINTS - (NS - 1) * CHUNK1
HALF4 = CHUNK1 // 2
HALF4_LAST = CHUNK1_LAST // 2
NBP = 576
ACC = 4 * NBP
COLCH = ACC // NS


def _rsqrt(x):
    xb = plsc.bitcast(x, jnp.int32)
    y = plsc.bitcast(jnp.int32(0x5F3759DF) - lax.shift_right_logical(xb, 1),
                     jnp.float32)
    hx = x * 0.5
    for _ in range(3):
        y = y * (1.5 - hx * y * y)
    return y


def _floxels_kernel(fx_hbm, fy_hbm, fz_hbm, clus_hbm, out_hbm,
                    fx_v, fy_v, fz_v, clus_v, hist_v, colchunk_v,
                    combchunk_v, out_v, dma_sem, part_sh, comb_sh):
    c = lax.axis_index("c")
    s = lax.axis_index("s")
    iota = lax.iota(jnp.int32, L)
    zeros16 = jnp.zeros((L,), jnp.float32)
    ones16 = jnp.ones((L,), jnp.float32)

    def phase1(base, npts, unroll):
        cps = [
            pltpu.async_copy(fx_hbm.at[pl.ds(base, npts)],
                             fx_v.at[pl.ds(0, npts)], dma_sem),
            pltpu.async_copy(fy_hbm.at[pl.ds(base, npts)],
                             fy_v.at[pl.ds(0, npts)], dma_sem),
            pltpu.async_copy(fz_hbm.at[pl.ds(base, npts)],
                             fz_v.at[pl.ds(0, npts)], dma_sem),
            pltpu.async_copy(clus_hbm.at[pl.ds(base, npts)],
                             clus_v.at[pl.ds(0, npts)], dma_sem),
        ]

        @plsc.parallel_loop(0, ACC, step=L, unroll=4)
        def _zero(b):
            hist_v[pl.ds(b, L)] = zeros16

        for cp in cps:
            cp.wait()

        @plsc.parallel_loop(0, npts, step=L, unroll=unroll)
        def _p1(b):
            cl = clus_v[pl.ds(b, L)]
            i4 = cl * 4
            plsc.addupdate_scatter(hist_v, [i4], fx_v[pl.ds(b, L)])
            plsc.addupdate_scatter(hist_v, [i4 + 1], fy_v[pl.ds(b, L)])
            plsc.addupdate_scatter(hist_v, [i4 + 2], fz_v[pl.ds(b, L)])
            plsc.addupdate_scatter(hist_v, [i4 + 3], ones16)

    pl.when(s < NS - 1)(lambda: phase1(s * CHUNK1, CHUNK1, 4))
    pl.when(s == NS - 1)(lambda: phase1((NS - 1) * CHUNK1, CHUNK1_LAST, 5))

    pltpu.sync_copy(hist_v, part_sh.at[pl.ds(s * ACC, ACC)])
    plsc.subcore_barrier()
    cps = [
        pltpu.async_copy(part_sh.at[pl.ds(l * ACC + s * COLCH, COLCH)],
                         colchunk_v.at[pl.ds(l * COLCH, COLCH)], dma_sem)
        for l in range(NS)
    ]
    for cp in cps:
        cp.wait()

    @plsc.parallel_loop(0, COLCH, step=L)
    def _p2(b):
        acc = colchunk_v[pl.ds(b, L)]
        for l in range(1, NS):
            acc = acc + colchunk_v[pl.ds(l * COLCH + b, L)]
        combchunk_v[pl.ds(b, L)] = acc

    pltpu.sync_copy(combchunk_v, comb_sh.at[pl.ds(s * COLCH, COLCH)])
    plsc.subcore_barrier()
    pltpu.sync_copy(comb_sh, hist_v)

    @plsc.parallel_loop(0, ACC, step=L, unroll=4)
    def _p3(b):
        num = hist_v[pl.ds(b, L)]
        den = plsc.load_gather(hist_v, [b + (iota | 3)])
        den = jnp.maximum(den, 1.0)
        hist_v[pl.ds(b, L)] = num / den

    def phase4(base, lb, npts, unroll):
        @plsc.parallel_loop(0, npts, step=L, unroll=unroll)
        def _p4(b):
            cl = clus_v[pl.ds(lb + b, L)]
            i4 = cl * 4
            dx = fx_v[pl.ds(lb + b, L)] - plsc.load_gather(hist_v, [i4])
            dy = fy_v[pl.ds(lb + b, L)] - plsc.load_gather(hist_v, [i4 + 1])
            dz = fz_v[pl.ds(lb + b, L)] - plsc.load_gather(hist_v, [i4 + 2])
            ss = dx * dx + dy * dy + dz * dz
            out_v[pl.ds(b, L)] = ss * _rsqrt(jnp.maximum(ss, 1e-30))

        pltpu.sync_copy(out_v.at[pl.ds(0, npts)],
                        out_hbm.at[pl.ds(base + lb, npts)])

    pl.when(s < NS - 1)(lambda: phase4(s * CHUNK1, c * HALF4, HALF4, 4))
    pl.when(s == NS - 1)(lambda: phase4((NS - 1) * CHUNK1, c * HALF4_LAST,
                                        HALF4_LAST, 5))


@jax.jit
def kernel(flow, clusters):
    fx = flow[:, 0]
    fy = flow[:, 1]
    fz = flow[:, 2]

    mesh = plsc.VectorSubcoreMesh(core_axis_name="c", subcore_axis_name="s")
    run = pl.kernel(
        _floxels_kernel,
        mesh=mesh,
        compiler_params=pltpu.CompilerParams(needs_layout_passes=False),
        out_type=jax.ShapeDtypeStruct((N_POINTS,), jnp.float32),
        scratch_types=[
            pltpu.VMEM((CHUNK1,), jnp.float32),
            pltpu.VMEM((CHUNK1,), jnp.float32),
            pltpu.VMEM((CHUNK1,), jnp.float32),
            pltpu.VMEM((CHUNK1,), jnp.int32),
            pltpu.VMEM((ACC,), jnp.float32),
            pltpu.VMEM((NS * COLCH,), jnp.float32),
            pltpu.VMEM((COLCH,), jnp.float32),
            pltpu.VMEM((HALF4,), jnp.float32),
            pltpu.SemaphoreType.DMA,
            pltpu.VMEM_SHARED((NS * ACC,), jnp.float32),
            pltpu.VMEM_SHARED((ACC,), jnp.float32),
        ],
    )
    return run(fx, fy, fz, clusters)
